# Initial kernel scaffold; baseline (speedup 1.0000x reference)
import jax
import jax.numpy as jnp
from jax.experimental import pallas as pl


def kernel(node_features, edge_indices, distances, W1, b1, W2, b2, Wt):
    raise NotImplementedError("write your pallas kernel here")


# speedup vs baseline: 2.0978x; 2.0978x over previous
import functools

import jax
import jax.numpy as jnp
from jax import lax
from jax.experimental import pallas as pl
from jax.experimental.pallas import tpu as pltpu
from jax.experimental.pallas import tpu_sc as plsc

N = 10000
E = 160000
D = 128
U = 32
G = 50
GP = 64
GAMMA = 10.0
MAXD = 30.0
LOG2 = 0.6931471805599453

NC = 2
NS = 16
NW = NC * NS
CH = 128
EP = 163840
CPW = EP // (NW * CH)
PW = EP // NW
NP = 10240
NPT = NP // NS

EB = 1280
NBLK = EP // EB
NREAL = E // EB


def _edge_body(d_ref, s_ref, wt_ref, cen_ref, w1_ref, b1_ref, r_ref, w2b_ref,
               b2m_ref, o_ref):
    i = pl.program_id(0)
    d = d_ref[...]
    cen = cen_ref[...]
    diff = d - cen
    df = jnp.exp(-GAMMA * diff * diff)
    h = jnp.dot(df, w1_ref[...], preferred_element_type=jnp.float32)
    h = h + b1_ref[...]
    t = jnp.dot(s_ref[...], wt_ref[...],
                preferred_element_type=jnp.float32)
    hrep = jnp.dot(h, r_ref[...], preferred_element_type=jnp.float32)
    ttile = jnp.tile(t, (1, U))
    p = hrep * ttile
    out = jnp.dot(p, w2b_ref[...], preferred_element_type=jnp.float32)
    out = out + jnp.dot(t, b2m_ref[...], preferred_element_type=jnp.float32)
    out = jnp.where(i < NREAL, out, 0.0)
    o_ref[...] = jnp.concatenate([out, jnp.zeros((EB, D - U), jnp.float32)],
                                 axis=1)


def _final_body(a_ref, b_ref, o_ref):
    x = a_ref[...] + b_ref[...]
    o_ref[...] = (jnp.maximum(x, 0.0)
                  + jnp.log(1.0 + jnp.exp(-jnp.abs(x))) - LOG2)


_MESH = plsc.VectorSubcoreMesh(core_axis_name="c", subcore_axis_name="s")


@functools.partial(
    pl.kernel,
    out_type=jax.ShapeDtypeStruct((EP, D), jnp.float32),
    mesh=_MESH,
    scratch_types=[
        pltpu.VMEM((CPW, CH), jnp.int32),
        pltpu.VMEM((CH, D), jnp.float32),
        pltpu.SemaphoreType.DMA,
    ],
)
def _sc_gather(x_hbm, idx_hbm, out_hbm, idx_v, buf, sem):
    wid = lax.axis_index("s") * NC + lax.axis_index("c")
    pltpu.sync_copy(idx_hbm.at[pl.ds(wid * CPW, CPW)], idx_v)

    def body(c, carry):
        pltpu.async_copy(x_hbm.at[idx_v.at[c]], buf, sem).wait()
        pltpu.sync_copy(buf, out_hbm.at[pl.ds(wid * PW + c * CH, CH)])
        return carry

    lax.fori_loop(0, CPW, body, 0)


@functools.partial(
    pl.kernel,
    out_type=jax.ShapeDtypeStruct((NC, NP, D), jnp.float32),
    mesh=_MESH,
    scratch_types=[
        pltpu.VMEM((CPW, CH), jnp.int32),
        pltpu.VMEM((CH, D), jnp.float32),
        pltpu.VMEM_SHARED((NP, D), jnp.float32),
        pltpu.SemaphoreType.DMA,
    ],
)
def _sc_scatter(f_hbm, dst_hbm, zero_hbm, out_hbm, idx_v, buf, acc, sem):
    cid = lax.axis_index("c")
    sid = lax.axis_index("s")
    wid = sid * NC + cid
    pltpu.sync_copy(zero_hbm, acc.at[pl.ds(sid * NPT, NPT)])
    plsc.subcore_barrier()
    pltpu.sync_copy(dst_hbm.at[pl.ds(wid * CPW, CPW)], idx_v)

    for c in range(CPW):
        pltpu.sync_copy(f_hbm.at[pl.ds(wid * PW + c * CH, CH)], buf)
        pltpu.sync_copy(buf, acc.at[idx_v.at[c]], add=True)
    plsc.subcore_barrier()
    pltpu.sync_copy(acc.at[pl.ds(sid * NPT, NPT)],
                    out_hbm.at[cid, pl.ds(sid * NPT, NPT)])


def kernel(node_features, edge_indices, distances, W1, b1, W2, b2, Wt):
    f32 = jnp.float32
    centers = jnp.linspace(0.0, MAXD, G).astype(f32)
    cen_pad = jnp.zeros((1, GP), f32).at[0, :G].set(centers)
    w1_pad = jnp.zeros((GP, U), f32).at[:G, :].set(W1)
    b1_row = b1.reshape(1, U)
    w2b = W2.reshape(U, U, U).transpose(0, 2, 1).reshape(U * U, U)
    b2m = b2.reshape(U, U).T
    rmat = jnp.kron(jnp.eye(U, dtype=f32), jnp.ones((1, U), f32))

    src = jnp.concatenate([edge_indices[0],
                           jnp.zeros((EP - E,), jnp.int32)]).reshape(-1, CH)
    dst = jnp.concatenate([edge_indices[1],
                           jnp.zeros((EP - E,), jnp.int32)]).reshape(-1, CH)
    dpad = jnp.concatenate([distances,
                            jnp.zeros((EP - E,), f32)]).reshape(EP, 1)
    zinit = jnp.zeros((NPT, D), f32)

    s_edges = _sc_gather(node_features, src)

    filtered = pl.pallas_call(
        _edge_body,
        grid=(NBLK,),
        in_specs=[
            pl.BlockSpec((EB, 1), lambda i: (i, 0)),
            pl.BlockSpec((EB, D), lambda i: (i, 0)),
            pl.BlockSpec((D, U), lambda i: (0, 0)),
            pl.BlockSpec((1, GP), lambda i: (0, 0)),
            pl.BlockSpec((GP, U), lambda i: (0, 0)),
            pl.BlockSpec((1, U), lambda i: (0, 0)),
            pl.BlockSpec((U, U * U), lambda i: (0, 0)),
            pl.BlockSpec((U * U, U), lambda i: (0, 0)),
            pl.BlockSpec((U, U), lambda i: (0, 0)),
        ],
        out_specs=pl.BlockSpec((EB, D), lambda i: (i, 0)),
        out_shape=jax.ShapeDtypeStruct((EP, D), f32),
    )(dpad, s_edges, Wt, cen_pad, w1_pad, b1_row, rmat, w2b, b2m)

    partials = _sc_scatter(filtered, dst, zinit)

    out = pl.pallas_call(
        _final_body,
        out_shape=jax.ShapeDtypeStruct((N, U), f32),
    )(partials[0, :N, :U], partials[1, :N, :U])
    return out


# speedup vs baseline: 2.7691x; 1.3200x over previous
import functools

import jax
import jax.numpy as jnp
from jax import lax
from jax.experimental import pallas as pl
from jax.experimental.pallas import tpu as pltpu
from jax.experimental.pallas import tpu_sc as plsc

N = 10000
E = 160000
D = 128
U = 32
G = 50
GP = 64
GAMMA = 10.0
MAXD = 30.0
LOG2 = 0.6931471805599453

NC = 2
NS = 16
NW = NC * NS
CH = 128
EP = 163840
CPW = EP // (NW * CH)
PW = EP // NW
NP = 10240
NPT = NP // NS

GQ = 10
NQ = CPW // GQ
SGRP = 8
NGRP = CPW // SGRP

EB = 1280
NBLK = EP // EB
NREAL = E // EB

_SC_PARAMS = pltpu.CompilerParams(use_tc_tiling_on_sc=False)


def _xform_body(nf_ref, wt_ref, o_ref):
    o_ref[...] = jnp.dot(nf_ref[...], wt_ref[...],
                         preferred_element_type=jnp.float32)


def _edge_body(d_ref, t_ref, cen_ref, w1_ref, b1_ref, r_ref, w2b_ref,
               b2m_ref, o_ref):
    i = pl.program_id(0)
    d = d_ref[...]
    cen = cen_ref[...]
    diff = d - cen
    df = jnp.exp(-GAMMA * diff * diff)
    h = jnp.dot(df, w1_ref[...], preferred_element_type=jnp.float32)
    h = h + b1_ref[...]
    t = t_ref[...]
    hrep = jnp.dot(h, r_ref[...], preferred_element_type=jnp.float32)
    ttile = jnp.tile(t, (1, U))
    p = hrep * ttile
    out = jnp.dot(p, w2b_ref[...], preferred_element_type=jnp.float32)
    out = out + jnp.dot(t, b2m_ref[...], preferred_element_type=jnp.float32)
    o_ref[...] = jnp.where(i < NREAL, out, 0.0)


def _final_body(a_ref, b_ref, o_ref):
    x = a_ref[...] + b_ref[...]
    o_ref[...] = (jnp.maximum(x, 0.0)
                  + jnp.log(1.0 + jnp.exp(-jnp.abs(x))) - LOG2)


_MESH = plsc.VectorSubcoreMesh(core_axis_name="c", subcore_axis_name="s")


@functools.partial(
    pl.kernel,
    out_type=jax.ShapeDtypeStruct((EP, U), jnp.float32),
    mesh=_MESH,
    scratch_types=[
        pltpu.VMEM((CPW, CH), jnp.int32),
        pltpu.VMEM((2, GQ * CH, U), jnp.float32),
        pltpu.SemaphoreType.DMA,
        pltpu.SemaphoreType.DMA,
    ],
    compiler_params=_SC_PARAMS,
)
def _sc_gather(x_hbm, idx_hbm, out_hbm, idx_v, bufs, gsem, csem):
    wid = lax.axis_index("s") * NC + lax.axis_index("c")
    pltpu.sync_copy(idx_hbm.at[pl.ds(wid * CPW, CPW)], idx_v)
    couts = [None, None]
    for q in range(NQ):
        buf = bufs.at[q % 2]
        if couts[q % 2] is not None:
            couts[q % 2].wait()
        descs = [
            pltpu.async_copy(
                x_hbm.at[idx_v.at[q * GQ + j]],
                buf.at[pl.ds(j * CH, CH)], gsem)
            for j in range(GQ)
        ]
        for dsc in descs:
            dsc.wait()
        couts[q % 2] = pltpu.async_copy(
            buf, out_hbm.at[pl.ds(wid * PW + q * GQ * CH, GQ * CH)], csem)
    couts[0].wait()
    couts[1].wait()


@functools.partial(
    pl.kernel,
    out_type=jax.ShapeDtypeStruct((NC, NP, U), jnp.float32),
    mesh=_MESH,
    scratch_types=[
        pltpu.VMEM((CPW, CH), jnp.int32),
        pltpu.VMEM((SGRP, CH, U), jnp.float32),
        pltpu.VMEM_SHARED((NP, U), jnp.float32),
        pltpu.SemaphoreType.DMA,
        pltpu.SemaphoreType.DMA,
    ],
    compiler_params=_SC_PARAMS,
)
def _sc_scatter(f_hbm, dst_hbm, zero_hbm, out_hbm, idx_v, bufs, acc,
                lsem, ssem):
    cid = lax.axis_index("c")
    sid = lax.axis_index("s")
    wid = sid * NC + cid
    pltpu.sync_copy(zero_hbm, acc.at[pl.ds(sid * NPT, NPT)])
    plsc.subcore_barrier()
    pltpu.sync_copy(dst_hbm.at[pl.ds(wid * CPW, CPW)], idx_v)

    for g in range(NGRP):
        base = wid * PW + g * SGRP * CH
        loads = [
            pltpu.async_copy(f_hbm.at[pl.ds(base + j * CH, CH)],
                             bufs.at[j], lsem)
            for j in range(SGRP)
        ]
        for dsc in loads:
            dsc.wait()
        scats = [
            pltpu.async_copy(bufs.at[j], acc.at[idx_v.at[g * SGRP + j]],
                             ssem, add=True)
            for j in range(SGRP)
        ]
        for dsc in scats:
            dsc.wait()
    plsc.subcore_barrier()
    pltpu.sync_copy(acc.at[pl.ds(sid * NPT, NPT)],
                    out_hbm.at[cid, pl.ds(sid * NPT, NPT)])


def kernel(node_features, edge_indices, distances, W1, b1, W2, b2, Wt):
    f32 = jnp.float32
    centers = jnp.linspace(0.0, MAXD, G).astype(f32)
    cen_pad = jnp.zeros((1, GP), f32).at[0, :G].set(centers)
    w1_pad = jnp.zeros((GP, U), f32).at[:G, :].set(W1)
    b1_row = b1.reshape(1, U)
    w2b = W2.reshape(U, U, U).transpose(0, 2, 1).reshape(U * U, U)
    b2m = b2.reshape(U, U).T
    rmat = jnp.kron(jnp.eye(U, dtype=f32), jnp.ones((1, U), f32))

    src = jnp.concatenate([edge_indices[0],
                           jnp.zeros((EP - E,), jnp.int32)]).reshape(-1, CH)
    dst = jnp.concatenate([edge_indices[1],
                           jnp.zeros((EP - E,), jnp.int32)]).reshape(-1, CH)
    dpad = jnp.concatenate([distances,
                            jnp.zeros((EP - E,), f32)]).reshape(EP, 1)
    zinit = jnp.zeros((NPT, U), f32)

    x = pl.pallas_call(
        _xform_body,
        out_shape=jax.ShapeDtypeStruct((N, U), f32),
    )(node_features, Wt)

    t_edges = _sc_gather(x, src)

    filtered = pl.pallas_call(
        _edge_body,
        grid=(NBLK,),
        in_specs=[
            pl.BlockSpec((EB, 1), lambda i: (i, 0)),
            pl.BlockSpec((EB, U), lambda i: (i, 0)),
            pl.BlockSpec((1, GP), lambda i: (0, 0)),
            pl.BlockSpec((GP, U), lambda i: (0, 0)),
            pl.BlockSpec((1, U), lambda i: (0, 0)),
            pl.BlockSpec((U, U * U), lambda i: (0, 0)),
            pl.BlockSpec((U * U, U), lambda i: (0, 0)),
            pl.BlockSpec((U, U), lambda i: (0, 0)),
        ],
        out_specs=pl.BlockSpec((EB, U), lambda i: (i, 0)),
        out_shape=jax.ShapeDtypeStruct((EP, U), f32),
    )(dpad, t_edges, cen_pad, w1_pad, b1_row, rmat, w2b, b2m)

    partials = _sc_scatter(filtered, dst, zinit)

    out = pl.pallas_call(
        _final_body,
        out_shape=jax.ShapeDtypeStruct((N, U), f32),
    )(partials[0, :N], partials[1, :N])
    return out


# speedup vs baseline: 2.8945x; 1.0453x over previous
import functools

import jax
import jax.numpy as jnp
from jax import lax
from jax.experimental import pallas as pl
from jax.experimental.pallas import tpu as pltpu
from jax.experimental.pallas import tpu_sc as plsc

N = 10000
E = 160000
D = 128
U = 32
G = 50
GP = 64
GAMMA = 10.0
MAXD = 30.0
LOG2 = 0.6931471805599453

NC = 2
NS = 16
NW = NC * NS
CH = 128
EP = 163840
CPW = EP // (NW * CH)
PW = EP // NW
NP = 10240
NPT = NP // NS

GQ = 10
NQ = CPW // GQ
SGRP = 8
NGRP = CPW // SGRP

EB = 2560
NBLK = EP // EB
NREAL = E // EB

_SC_PARAMS = pltpu.CompilerParams(use_tc_tiling_on_sc=False)


def _xform_body(nf_ref, wt_ref, o_ref):
    o_ref[...] = jnp.dot(nf_ref[...], wt_ref[...],
                         preferred_element_type=jnp.float32)


def _edge_body(d_ref, t_ref, cen_ref, w1_ref, b1_ref, r_ref, w2b_ref,
               b2m_ref, o_ref):
    i = pl.program_id(0)
    d = d_ref[...]
    cen = cen_ref[...]
    diff = d - cen
    df = jnp.exp(-GAMMA * diff * diff)
    h = jnp.dot(df, w1_ref[...], preferred_element_type=jnp.float32)
    h = h + b1_ref[...]
    t = t_ref[...]
    hrep = jnp.dot(h, r_ref[...], preferred_element_type=jnp.float32)
    ttile = jnp.tile(t, (1, U))
    p = hrep * ttile
    out = jnp.dot(p, w2b_ref[...], preferred_element_type=jnp.float32)
    out = out + jnp.dot(t, b2m_ref[...], preferred_element_type=jnp.float32)
    row = i * EB + lax.broadcasted_iota(jnp.int32, (EB, 1), 0)
    o_ref[...] = jnp.where(row < E, out, 0.0)


def _final_body(a_ref, b_ref, o_ref):
    x = a_ref[...] + b_ref[...]
    o_ref[...] = (jnp.maximum(x, 0.0)
                  + jnp.log(1.0 + jnp.exp(-jnp.abs(x))) - LOG2)


_MESH = plsc.VectorSubcoreMesh(core_axis_name="c", subcore_axis_name="s")


@functools.partial(
    pl.kernel,
    out_type=jax.ShapeDtypeStruct((EP, U), jnp.float32),
    mesh=_MESH,
    scratch_types=[
        pltpu.VMEM((CPW, CH), jnp.int32),
        pltpu.VMEM((2, GQ * CH, U), jnp.float32),
        pltpu.SemaphoreType.DMA,
        pltpu.SemaphoreType.DMA,
    ],
    compiler_params=_SC_PARAMS,
)
def _sc_gather(x_hbm, idx_hbm, out_hbm, idx_v, bufs, gsem, csem):
    wid = lax.axis_index("s") * NC + lax.axis_index("c")
    pltpu.sync_copy(idx_hbm.at[pl.ds(wid * CPW, CPW)], idx_v)
    couts = [None, None]
    for q in range(NQ):
        buf = bufs.at[q % 2]
        if couts[q % 2] is not None:
            couts[q % 2].wait()
        descs = [
            pltpu.async_copy(
                x_hbm.at[idx_v.at[q * GQ + j]],
                buf.at[pl.ds(j * CH, CH)], gsem)
            for j in range(GQ)
        ]
        for dsc in descs:
            dsc.wait()
        couts[q % 2] = pltpu.async_copy(
            buf, out_hbm.at[pl.ds(wid * PW + q * GQ * CH, GQ * CH)], csem)
    couts[0].wait()
    couts[1].wait()


@functools.partial(
    pl.kernel,
    out_type=jax.ShapeDtypeStruct((NC, NP, U), jnp.float32),
    mesh=_MESH,
    scratch_types=[
        pltpu.VMEM((CPW, CH), jnp.int32),
        pltpu.VMEM((SGRP, CH, U), jnp.float32),
        pltpu.VMEM_SHARED((NP, U), jnp.float32),
        pltpu.SemaphoreType.DMA,
        pltpu.SemaphoreType.DMA,
    ],
    compiler_params=_SC_PARAMS,
)
def _sc_scatter(f_hbm, dst_hbm, zero_hbm, out_hbm, idx_v, bufs, acc,
                lsem, ssem):
    cid = lax.axis_index("c")
    sid = lax.axis_index("s")
    wid = sid * NC + cid
    pltpu.sync_copy(zero_hbm, acc.at[pl.ds(sid * NPT, NPT)])
    plsc.subcore_barrier()
    pltpu.sync_copy(dst_hbm.at[pl.ds(wid * CPW, CPW)], idx_v)

    for g in range(NGRP):
        base = wid * PW + g * SGRP * CH
        loads = [
            pltpu.async_copy(f_hbm.at[pl.ds(base + j * CH, CH)],
                             bufs.at[j], lsem)
            for j in range(SGRP)
        ]
        for dsc in loads:
            dsc.wait()
        scats = [
            pltpu.async_copy(bufs.at[j], acc.at[idx_v.at[g * SGRP + j]],
                             ssem, add=True)
            for j in range(SGRP)
        ]
        for dsc in scats:
            dsc.wait()
    plsc.subcore_barrier()
    pltpu.sync_copy(acc.at[pl.ds(sid * NPT, NPT)],
                    out_hbm.at[cid, pl.ds(sid * NPT, NPT)])


def kernel(node_features, edge_indices, distances, W1, b1, W2, b2, Wt):
    f32 = jnp.float32
    centers = jnp.linspace(0.0, MAXD, G).astype(f32)
    cen_pad = jnp.zeros((1, GP), f32).at[0, :G].set(centers)
    w1_pad = jnp.zeros((GP, U), f32).at[:G, :].set(W1)
    b1_row = b1.reshape(1, U)
    w2b = W2.reshape(U, U, U).transpose(0, 2, 1).reshape(U * U, U)
    b2m = b2.reshape(U, U).T
    rmat = jnp.kron(jnp.eye(U, dtype=f32), jnp.ones((1, U), f32))

    src = jnp.concatenate([edge_indices[0],
                           jnp.zeros((EP - E,), jnp.int32)]).reshape(-1, CH)
    dst = jnp.concatenate([edge_indices[1],
                           jnp.zeros((EP - E,), jnp.int32)]).reshape(-1, CH)
    dpad = jnp.concatenate([distances,
                            jnp.zeros((EP - E,), f32)]).reshape(EP, 1)
    zinit = jnp.zeros((NPT, U), f32)

    x = pl.pallas_call(
        _xform_body,
        out_shape=jax.ShapeDtypeStruct((N, U), f32),
    )(node_features, Wt)

    t_edges = _sc_gather(x, src)

    filtered = pl.pallas_call(
        _edge_body,
        grid=(NBLK,),
        in_specs=[
            pl.BlockSpec((EB, 1), lambda i: (i, 0)),
            pl.BlockSpec((EB, U), lambda i: (i, 0)),
            pl.BlockSpec((1, GP), lambda i: (0, 0)),
            pl.BlockSpec((GP, U), lambda i: (0, 0)),
            pl.BlockSpec((1, U), lambda i: (0, 0)),
            pl.BlockSpec((U, U * U), lambda i: (0, 0)),
            pl.BlockSpec((U * U, U), lambda i: (0, 0)),
            pl.BlockSpec((U, U), lambda i: (0, 0)),
        ],
        out_specs=pl.BlockSpec((EB, U), lambda i: (i, 0)),
        out_shape=jax.ShapeDtypeStruct((EP, U), f32),
    )(dpad, t_edges, cen_pad, w1_pad, b1_row, rmat, w2b, b2m)

    partials = _sc_scatter(filtered, dst, zinit)

    out = pl.pallas_call(
        _final_body,
        out_shape=jax.ShapeDtypeStruct((N, U), f32),
    )(partials[0, :N], partials[1, :N])
    return out


# speedup vs baseline: 2.9552x; 1.0210x over previous
import functools

import jax
import jax.numpy as jnp
from jax import lax
from jax.experimental import pallas as pl
from jax.experimental.pallas import tpu as pltpu
from jax.experimental.pallas import tpu_sc as plsc

N = 10000
E = 160000
D = 128
U = 32
G = 50
GP = 64
GAMMA = 10.0
MAXD = 30.0
LOG2 = 0.6931471805599453

NC = 2
NS = 16
NW = NC * NS
CH = 128
EP = 163840
CPW = EP // (NW * CH)
PW = EP // NW
NP = 10240
NPT = NP // NS

GQ = 10
NQ = CPW // GQ
SGRP = 8
NGRP = CPW // SGRP

EB = 2560
NBLK = EP // EB
NREAL = E // EB

_SC_PARAMS = pltpu.CompilerParams(use_tc_tiling_on_sc=False)


def _xform_body(nf_ref, wt_ref, o_ref):
    o_ref[...] = jnp.dot(nf_ref[...], wt_ref[...],
                         preferred_element_type=jnp.float32)


def _edge_body(d_ref, t_ref, cen_ref, w1_ref, b1_ref, r_ref, w2b_ref,
               b2m_ref, o_ref):
    i = pl.program_id(0)
    d = d_ref[...]
    cen = cen_ref[...]
    diff = d - cen
    df = jnp.exp(-GAMMA * diff * diff)
    t = t_ref[...]
    hrep = (jnp.dot(df.astype(jnp.bfloat16), r_ref[...],
                    preferred_element_type=jnp.float32)
            + b1_ref[...]).astype(jnp.bfloat16)
    ttile = jnp.tile(t.astype(jnp.bfloat16), (1, U))
    p = hrep * ttile
    out = jnp.dot(p, w2b_ref[...], preferred_element_type=jnp.float32)
    out = out + jnp.dot(t, b2m_ref[...], preferred_element_type=jnp.float32)
    row = i * EB + lax.broadcasted_iota(jnp.int32, (EB, 1), 0)
    o_ref[...] = jnp.where(row < E, out, 0.0)


def _final_body(a_ref, b_ref, o_ref):
    x = a_ref[...] + b_ref[...]
    o_ref[...] = (jnp.maximum(x, 0.0)
                  + jnp.log(1.0 + jnp.exp(-jnp.abs(x))) - LOG2)


_MESH = plsc.VectorSubcoreMesh(core_axis_name="c", subcore_axis_name="s")


@functools.partial(
    pl.kernel,
    out_type=jax.ShapeDtypeStruct((EP, U), jnp.float32),
    mesh=_MESH,
    scratch_types=[
        pltpu.VMEM((CPW, CH), jnp.int32),
        pltpu.VMEM((2, GQ * CH, U), jnp.float32),
        pltpu.SemaphoreType.DMA,
        pltpu.SemaphoreType.DMA,
    ],
    compiler_params=_SC_PARAMS,
)
def _sc_gather(x_hbm, idx_hbm, out_hbm, idx_v, bufs, gsem, csem):
    wid = lax.axis_index("s") * NC + lax.axis_index("c")
    pltpu.sync_copy(idx_hbm.at[pl.ds(wid * CPW, CPW)], idx_v)
    couts = [None, None]
    for q in range(NQ):
        buf = bufs.at[q % 2]
        if couts[q % 2] is not None:
            couts[q % 2].wait()
        descs = [
            pltpu.async_copy(
                x_hbm.at[idx_v.at[q * GQ + j]],
                buf.at[pl.ds(j * CH, CH)], gsem)
            for j in range(GQ)
        ]
        for dsc in descs:
            dsc.wait()
        couts[q % 2] = pltpu.async_copy(
            buf, out_hbm.at[pl.ds(wid * PW + q * GQ * CH, GQ * CH)], csem)
    couts[0].wait()
    couts[1].wait()


@functools.partial(
    pl.kernel,
    out_type=jax.ShapeDtypeStruct((NC, NP, U), jnp.float32),
    mesh=_MESH,
    scratch_types=[
        pltpu.VMEM((CPW, CH), jnp.int32),
        pltpu.VMEM((SGRP, CH, U), jnp.float32),
        pltpu.VMEM_SHARED((NP, U), jnp.float32),
        pltpu.SemaphoreType.DMA,
        pltpu.SemaphoreType.DMA,
    ],
    compiler_params=_SC_PARAMS,
)
def _sc_scatter(f_hbm, dst_hbm, zero_hbm, out_hbm, idx_v, bufs, acc,
                lsem, ssem):
    cid = lax.axis_index("c")
    sid = lax.axis_index("s")
    wid = sid * NC + cid
    pltpu.sync_copy(zero_hbm, acc.at[pl.ds(sid * NPT, NPT)])
    plsc.subcore_barrier()
    pltpu.sync_copy(dst_hbm.at[pl.ds(wid * CPW, CPW)], idx_v)

    for g in range(NGRP):
        base = wid * PW + g * SGRP * CH
        loads = [
            pltpu.async_copy(f_hbm.at[pl.ds(base + j * CH, CH)],
                             bufs.at[j], lsem)
            for j in range(SGRP)
        ]
        for dsc in loads:
            dsc.wait()
        scats = [
            pltpu.async_copy(bufs.at[j], acc.at[idx_v.at[g * SGRP + j]],
                             ssem, add=True)
            for j in range(SGRP)
        ]
        for dsc in scats:
            dsc.wait()
    plsc.subcore_barrier()
    pltpu.sync_copy(acc.at[pl.ds(sid * NPT, NPT)],
                    out_hbm.at[cid, pl.ds(sid * NPT, NPT)])


def kernel(node_features, edge_indices, distances, W1, b1, W2, b2, Wt):
    f32 = jnp.float32
    centers = jnp.linspace(0.0, MAXD, G).astype(f32)
    cen_pad = jnp.zeros((1, GP), f32).at[0, :G].set(centers)
    w1_pad = jnp.zeros((GP, U), f32).at[:G, :].set(W1)
    b1_row = b1.reshape(1, U)
    w2b = W2.reshape(U, U, U).transpose(0, 2, 1).reshape(U * U, U)
    b2m = b2.reshape(U, U).T
    rmat = jnp.repeat(w1_pad, U, axis=1).astype(jnp.bfloat16)
    b1_rep = jnp.repeat(b1_row, U, axis=1)

    src = jnp.concatenate([edge_indices[0],
                           jnp.zeros((EP - E,), jnp.int32)]).reshape(-1, CH)
    dst = jnp.concatenate([edge_indices[1],
                           jnp.zeros((EP - E,), jnp.int32)]).reshape(-1, CH)
    dpad = jnp.concatenate([distances,
                            jnp.zeros((EP - E,), f32)]).reshape(EP, 1)
    zinit = jnp.zeros((NPT, U), f32)

    x = pl.pallas_call(
        _xform_body,
        out_shape=jax.ShapeDtypeStruct((N, U), f32),
    )(node_features, Wt)

    t_edges = _sc_gather(x, src)

    filtered = pl.pallas_call(
        _edge_body,
        grid=(NBLK,),
        in_specs=[
            pl.BlockSpec((EB, 1), lambda i: (i, 0)),
            pl.BlockSpec((EB, U), lambda i: (i, 0)),
            pl.BlockSpec((1, GP), lambda i: (0, 0)),
            pl.BlockSpec((GP, U), lambda i: (0, 0)),
            pl.BlockSpec((1, U * U), lambda i: (0, 0)),
            pl.BlockSpec((GP, U * U), lambda i: (0, 0)),
            pl.BlockSpec((U * U, U), lambda i: (0, 0)),
            pl.BlockSpec((U, U), lambda i: (0, 0)),
        ],
        out_specs=pl.BlockSpec((EB, U), lambda i: (i, 0)),
        out_shape=jax.ShapeDtypeStruct((EP, U), f32),
    )(dpad, t_edges, cen_pad, w1_pad, b1_rep, rmat,
      w2b.astype(jnp.bfloat16), b2m)

    partials = _sc_scatter(filtered, dst, zinit)

    out = pl.pallas_call(
        _final_body,
        out_shape=jax.ShapeDtypeStruct((N, U), f32),
    )(partials[0, :N], partials[1, :N])
    return out


# speedup vs baseline: 3.1136x; 1.0536x over previous
import functools

import jax
import jax.numpy as jnp
from jax import lax
from jax.experimental import pallas as pl
from jax.experimental.pallas import tpu as pltpu
from jax.experimental.pallas import tpu_sc as plsc

N = 10000
E = 160000
D = 128
U = 32
G = 50
GP = 64
GAMMA = 10.0
MAXD = 30.0
LOG2 = 0.6931471805599453

NC = 2
NS = 16
NW = NC * NS
CH = 128
EP = 163840
CPW = EP // (NW * CH)
PW = EP // NW
NP = 10240
NPT = NP // NS

GQ = 10
NQ = CPW // GQ
SGRP = 8
NGRP = CPW // SGRP

EB = 2560
NBLK = EP // EB
NREAL = E // EB

_SC_PARAMS = pltpu.CompilerParams(use_tc_tiling_on_sc=False)


def _xform_body(nf_ref, wt_ref, o_ref):
    o_ref[...] = jnp.dot(nf_ref[...], wt_ref[...],
                         preferred_element_type=jnp.float32)


def _edge_body(d_ref, t_ref, cen_ref, r_ref, b1_ref, w2b_ref,
               b2m_ref, o_ref):
    i = pl.program_id(0)
    dt = d_ref[...]
    cen = cen_ref[...]
    diff = dt - cen
    dft = jnp.exp(-GAMMA * diff * diff)
    t = t_ref[...]
    hrep = (lax.dot_general(dft.astype(jnp.bfloat16), r_ref[...],
                            (((0,), (0,)), ((), ())),
                            preferred_element_type=jnp.float32)
            + b1_ref[...]).astype(jnp.bfloat16)
    ttile = jnp.tile(t.astype(jnp.bfloat16), (1, U))
    p = hrep * ttile
    out = jnp.dot(p, w2b_ref[...], preferred_element_type=jnp.float32)
    out = out + jnp.dot(t, b2m_ref[...], preferred_element_type=jnp.float32)
    row = i * EB + lax.broadcasted_iota(jnp.int32, (EB, 1), 0)
    o_ref[...] = jnp.where(row < E, out, 0.0)


def _final_body(a_ref, b_ref, o_ref):
    x = a_ref[...] + b_ref[...]
    o_ref[...] = (jnp.maximum(x, 0.0)
                  + jnp.log(1.0 + jnp.exp(-jnp.abs(x))) - LOG2)


_MESH = plsc.VectorSubcoreMesh(core_axis_name="c", subcore_axis_name="s")


@functools.partial(
    pl.kernel,
    out_type=jax.ShapeDtypeStruct((EP, U), jnp.float32),
    mesh=_MESH,
    scratch_types=[
        pltpu.VMEM((CPW, CH), jnp.int32),
        pltpu.VMEM((2, GQ * CH, U), jnp.float32),
        pltpu.SemaphoreType.DMA,
        pltpu.SemaphoreType.DMA,
    ],
    compiler_params=_SC_PARAMS,
)
def _sc_gather(x_hbm, idx_hbm, out_hbm, idx_v, bufs, gsem, csem):
    wid = lax.axis_index("s") * NC + lax.axis_index("c")
    pltpu.sync_copy(idx_hbm.at[pl.ds(wid * CPW, CPW)], idx_v)
    couts = [None, None]
    for q in range(NQ):
        buf = bufs.at[q % 2]
        if couts[q % 2] is not None:
            couts[q % 2].wait()
        descs = [
            pltpu.async_copy(
                x_hbm.at[idx_v.at[q * GQ + j]],
                buf.at[pl.ds(j * CH, CH)], gsem)
            for j in range(GQ)
        ]
        for dsc in descs:
            dsc.wait()
        couts[q % 2] = pltpu.async_copy(
            buf, out_hbm.at[pl.ds(wid * PW + q * GQ * CH, GQ * CH)], csem)
    couts[0].wait()
    couts[1].wait()


@functools.partial(
    pl.kernel,
    out_type=jax.ShapeDtypeStruct((NC, NP, U), jnp.float32),
    mesh=_MESH,
    scratch_types=[
        pltpu.VMEM((CPW, CH), jnp.int32),
        pltpu.VMEM((SGRP, CH, U), jnp.float32),
        pltpu.VMEM_SHARED((NP, U), jnp.float32),
        pltpu.SemaphoreType.DMA,
        pltpu.SemaphoreType.DMA,
    ],
    compiler_params=_SC_PARAMS,
)
def _sc_scatter(f_hbm, dst_hbm, zero_hbm, out_hbm, idx_v, bufs, acc,
                lsem, ssem):
    cid = lax.axis_index("c")
    sid = lax.axis_index("s")
    wid = sid * NC + cid
    pltpu.sync_copy(zero_hbm, acc.at[pl.ds(sid * NPT, NPT)])
    plsc.subcore_barrier()
    pltpu.sync_copy(dst_hbm.at[pl.ds(wid * CPW, CPW)], idx_v)

    for g in range(NGRP):
        base = wid * PW + g * SGRP * CH
        loads = [
            pltpu.async_copy(f_hbm.at[pl.ds(base + j * CH, CH)],
                             bufs.at[j], lsem)
            for j in range(SGRP)
        ]
        for dsc in loads:
            dsc.wait()
        scats = [
            pltpu.async_copy(bufs.at[j], acc.at[idx_v.at[g * SGRP + j]],
                             ssem, add=True)
            for j in range(SGRP)
        ]
        for dsc in scats:
            dsc.wait()
    plsc.subcore_barrier()
    pltpu.sync_copy(acc.at[pl.ds(sid * NPT, NPT)],
                    out_hbm.at[cid, pl.ds(sid * NPT, NPT)])


def kernel(node_features, edge_indices, distances, W1, b1, W2, b2, Wt):
    f32 = jnp.float32
    centers = jnp.linspace(0.0, MAXD, G).astype(f32)
    cen_col = jnp.zeros((GP, 1), f32).at[:G, 0].set(centers)
    w1_pad = jnp.zeros((GP, U), f32).at[:G, :].set(W1)
    b1_row = b1.reshape(1, U)
    w2b = W2.reshape(U, U, U).transpose(0, 2, 1).reshape(U * U, U)
    b2m = b2.reshape(U, U).T
    rmat = jnp.repeat(w1_pad, U, axis=1).astype(jnp.bfloat16)
    b1_rep = jnp.repeat(b1_row, U, axis=1)

    src = jnp.concatenate([edge_indices[0],
                           jnp.zeros((EP - E,), jnp.int32)]).reshape(-1, CH)
    dst = jnp.concatenate([edge_indices[1],
                           jnp.zeros((EP - E,), jnp.int32)]).reshape(-1, CH)
    dpad = jnp.concatenate([distances, jnp.zeros((EP - E,), f32)])
    zinit = jnp.zeros((NPT, U), f32)

    x = pl.pallas_call(
        _xform_body,
        out_shape=jax.ShapeDtypeStruct((N, U), f32),
    )(node_features, Wt)

    t_edges = _sc_gather(x, src)

    filtered = pl.pallas_call(
        _edge_body,
        grid=(NBLK,),
        in_specs=[
            pl.BlockSpec((1, EB), lambda i: (0, i)),
            pl.BlockSpec((EB, U), lambda i: (i, 0)),
            pl.BlockSpec((GP, 1), lambda i: (0, 0)),
            pl.BlockSpec((GP, U * U), lambda i: (0, 0)),
            pl.BlockSpec((1, U * U), lambda i: (0, 0)),
            pl.BlockSpec((U * U, U), lambda i: (0, 0)),
            pl.BlockSpec((U, U), lambda i: (0, 0)),
        ],
        out_specs=pl.BlockSpec((EB, U), lambda i: (i, 0)),
        out_shape=jax.ShapeDtypeStruct((EP, U), f32),
    )(dpad.reshape(1, EP), t_edges, cen_col, rmat, b1_rep,
      w2b.astype(jnp.bfloat16), b2m)

    partials = _sc_scatter(filtered, dst, zinit)

    out = pl.pallas_call(
        _final_body,
        out_shape=jax.ShapeDtypeStruct((N, U), f32),
    )(partials[0, :N], partials[1, :N])
    return out


# speedup vs baseline: 3.1813x; 1.0218x over previous
import functools

import jax
import jax.numpy as jnp
from jax import lax
from jax.experimental import pallas as pl
from jax.experimental.pallas import tpu as pltpu
from jax.experimental.pallas import tpu_sc as plsc

N = 10000
E = 160000
D = 128
U = 32
G = 50
GP = 64
GAMMA = 10.0
MAXD = 30.0
LOG2 = 0.6931471805599453

NC = 2
NS = 16
NW = NC * NS
CH = 128
EP = 163840
CPW = EP // (NW * CH)
PW = EP // NW
NP = 10240
NPT = NP // NS

GQ = 10
NQ = CPW // GQ
SGRP = 8
NGRP = CPW // SGRP

EB = 2560
NBLK = EP // EB
NREAL = E // EB

_SC_PARAMS = pltpu.CompilerParams(use_tc_tiling_on_sc=False)


def _xform_body(nf_ref, wt_ref, o_ref):
    o_ref[...] = jnp.dot(nf_ref[...], wt_ref[...],
                         preferred_element_type=jnp.float32
                         ).astype(jnp.bfloat16)


def _edge_body(d_ref, t_ref, cen_ref, r_ref, b1_ref, w2b_ref,
               b2m_ref, o_ref):
    i = pl.program_id(0)
    dt = d_ref[...]
    cen = cen_ref[...]
    diff = dt - cen
    dft = jnp.exp(-GAMMA * diff * diff)
    tb = t_ref[...]
    t = tb.astype(jnp.float32)
    hrep = (lax.dot_general(dft.astype(jnp.bfloat16), r_ref[...],
                            (((0,), (0,)), ((), ())),
                            preferred_element_type=jnp.float32)
            + b1_ref[...]).astype(jnp.bfloat16)
    ttile = jnp.tile(tb, (1, U))
    p = hrep * ttile
    out = jnp.dot(p, w2b_ref[...], preferred_element_type=jnp.float32)
    out = out + jnp.dot(t, b2m_ref[...], preferred_element_type=jnp.float32)
    row = i * EB + lax.broadcasted_iota(jnp.int32, (EB, 1), 0)
    o_ref[...] = jnp.where(row < E, out, 0.0)


def _final_body(a_ref, b_ref, o_ref):
    x = a_ref[...] + b_ref[...]
    o_ref[...] = (jnp.maximum(x, 0.0)
                  + jnp.log(1.0 + jnp.exp(-jnp.abs(x))) - LOG2)


_MESH = plsc.VectorSubcoreMesh(core_axis_name="c", subcore_axis_name="s")


@functools.partial(
    pl.kernel,
    out_type=jax.ShapeDtypeStruct((EP, U), jnp.bfloat16),
    mesh=_MESH,
    scratch_types=[
        pltpu.VMEM((CPW, CH), jnp.int32),
        pltpu.VMEM((2, GQ * CH, U), jnp.bfloat16),
        pltpu.SemaphoreType.DMA,
        pltpu.SemaphoreType.DMA,
    ],
    compiler_params=_SC_PARAMS,
)
def _sc_gather(x_hbm, idx_hbm, out_hbm, idx_v, bufs, gsem, csem):
    wid = lax.axis_index("s") * NC + lax.axis_index("c")
    pltpu.sync_copy(idx_hbm.at[pl.ds(wid * CPW, CPW)], idx_v)
    couts = [None, None]
    for q in range(NQ):
        buf = bufs.at[q % 2]
        if couts[q % 2] is not None:
            couts[q % 2].wait()
        descs = [
            pltpu.async_copy(
                x_hbm.at[idx_v.at[q * GQ + j]],
                buf.at[pl.ds(j * CH, CH)], gsem)
            for j in range(GQ)
        ]
        for dsc in descs:
            dsc.wait()
        couts[q % 2] = pltpu.async_copy(
            buf, out_hbm.at[pl.ds(wid * PW + q * GQ * CH, GQ * CH)], csem)
    couts[0].wait()
    couts[1].wait()


@functools.partial(
    pl.kernel,
    out_type=jax.ShapeDtypeStruct((NC, NP, U), jnp.float32),
    mesh=_MESH,
    scratch_types=[
        pltpu.VMEM((CPW, CH), jnp.int32),
        pltpu.VMEM((SGRP, CH, U), jnp.float32),
        pltpu.VMEM_SHARED((NP, U), jnp.float32),
        pltpu.SemaphoreType.DMA,
        pltpu.SemaphoreType.DMA,
    ],
    compiler_params=_SC_PARAMS,
)
def _sc_scatter(f_hbm, dst_hbm, zero_hbm, out_hbm, idx_v, bufs, acc,
                lsem, ssem):
    cid = lax.axis_index("c")
    sid = lax.axis_index("s")
    wid = sid * NC + cid
    pltpu.sync_copy(zero_hbm, acc.at[pl.ds(sid * NPT, NPT)])
    plsc.subcore_barrier()
    pltpu.sync_copy(dst_hbm.at[pl.ds(wid * CPW, CPW)], idx_v)

    for g in range(NGRP):
        base = wid * PW + g * SGRP * CH
        loads = [
            pltpu.async_copy(f_hbm.at[pl.ds(base + j * CH, CH)],
                             bufs.at[j], lsem)
            for j in range(SGRP)
        ]
        for dsc in loads:
            dsc.wait()
        scats = [
            pltpu.async_copy(bufs.at[j], acc.at[idx_v.at[g * SGRP + j]],
                             ssem, add=True)
            for j in range(SGRP)
        ]
        for dsc in scats:
            dsc.wait()
    plsc.subcore_barrier()
    pltpu.sync_copy(acc.at[pl.ds(sid * NPT, NPT)],
                    out_hbm.at[cid, pl.ds(sid * NPT, NPT)])


def kernel(node_features, edge_indices, distances, W1, b1, W2, b2, Wt):
    f32 = jnp.float32
    centers = jnp.linspace(0.0, MAXD, G).astype(f32)
    cen_col = jnp.zeros((GP, 1), f32).at[:G, 0].set(centers)
    w1_pad = jnp.zeros((GP, U), f32).at[:G, :].set(W1)
    b1_row = b1.reshape(1, U)
    w2b = W2.reshape(U, U, U).transpose(0, 2, 1).reshape(U * U, U)
    b2m = b2.reshape(U, U).T
    rmat = jnp.repeat(w1_pad, U, axis=1).astype(jnp.bfloat16)
    b1_rep = jnp.repeat(b1_row, U, axis=1)

    src = jnp.concatenate([edge_indices[0],
                           jnp.zeros((EP - E,), jnp.int32)]).reshape(-1, CH)
    dst = jnp.concatenate([edge_indices[1],
                           jnp.zeros((EP - E,), jnp.int32)]).reshape(-1, CH)
    dpad = jnp.concatenate([distances, jnp.zeros((EP - E,), f32)])
    zinit = jnp.zeros((NPT, U), f32)

    x = pl.pallas_call(
        _xform_body,
        out_shape=jax.ShapeDtypeStruct((N, U), jnp.bfloat16),
    )(node_features, Wt)

    t_edges = _sc_gather(x, src)

    filtered = pl.pallas_call(
        _edge_body,
        grid=(NBLK,),
        in_specs=[
            pl.BlockSpec((1, EB), lambda i: (0, i)),
            pl.BlockSpec((EB, U), lambda i: (i, 0)),
            pl.BlockSpec((GP, 1), lambda i: (0, 0)),
            pl.BlockSpec((GP, U * U), lambda i: (0, 0)),
            pl.BlockSpec((1, U * U), lambda i: (0, 0)),
            pl.BlockSpec((U * U, U), lambda i: (0, 0)),
            pl.BlockSpec((U, U), lambda i: (0, 0)),
        ],
        out_specs=pl.BlockSpec((EB, U), lambda i: (i, 0)),
        out_shape=jax.ShapeDtypeStruct((EP, U), f32),
    )(dpad.reshape(1, EP), t_edges, cen_col, rmat, b1_rep,
      w2b.astype(jnp.bfloat16), b2m)

    partials = _sc_scatter(filtered, dst, zinit)

    out = pl.pallas_call(
        _final_body,
        out_shape=jax.ShapeDtypeStruct((N, U), f32),
    )(partials[0, :N], partials[1, :N])
    return out


# speedup vs baseline: 3.8927x; 1.2236x over previous
import functools

import jax
import jax.numpy as jnp
from jax import lax
from jax.experimental import pallas as pl
from jax.experimental.pallas import tpu as pltpu
from jax.experimental.pallas import tpu_sc as plsc

N = 10000
E = 160000
D = 128
U = 32
G = 50
GP = 64
GAMMA = 10.0
MAXD = 30.0
LOG2 = 0.6931471805599453

NC = 2
NS = 16
NW = NC * NS
CH = 128
EP = 163840
CPW = EP // (NW * CH)
PW = EP // NW
NP = 10240
NPT = NP // NS

GQ = 10
NQ = CPW // GQ
SGRP = 8
NGRP = CPW // SGRP

EB = 2560
EB4 = EB // 4
NBLK = EP // EB
NREAL = E // EB

_SC_PARAMS = pltpu.CompilerParams(use_tc_tiling_on_sc=False)


def _xform_body(nf_ref, wt_ref, o_ref):
    o_ref[...] = jnp.dot(nf_ref[...], wt_ref[...],
                         preferred_element_type=jnp.float32)


def _edge_body(d_ref, t_ref, cen_ref, r_ref, b1_ref, w2b_ref,
               b2m_ref, o_ref):
    i = pl.program_id(0)
    d4 = d_ref[...]
    cen = cen_ref[...]
    t4 = t_ref[...]
    row4 = i * EB4 + lax.broadcasted_iota(jnp.int32, (EB4, 1), 0)
    mask = row4 < E // 4
    outs = []
    for q in range(4):
        dt = d4[q:q + 1, :]
        diff = dt - cen
        dft = jnp.exp(-GAMMA * diff * diff)
        t = t4[:, q * U:(q + 1) * U]
        tb = t.astype(jnp.bfloat16)
        hrep = (lax.dot_general(dft.astype(jnp.bfloat16), r_ref[...],
                                (((0,), (0,)), ((), ())),
                                preferred_element_type=jnp.float32)
                + b1_ref[...]).astype(jnp.bfloat16)
        ttile = jnp.tile(tb, (1, U))
        p = hrep * ttile
        out = jnp.dot(p, w2b_ref[...], preferred_element_type=jnp.float32)
        out = out + jnp.dot(t, b2m_ref[...],
                            preferred_element_type=jnp.float32)
        outs.append(jnp.where(mask, out, 0.0))
    o_ref[...] = jnp.concatenate(outs, axis=1)


def _final_body(a_ref, b_ref, o_ref):
    x = a_ref[...] + b_ref[...]
    o_ref[...] = (jnp.maximum(x, 0.0)
                  + jnp.log(1.0 + jnp.exp(-jnp.abs(x))) - LOG2)


_MESH = plsc.VectorSubcoreMesh(core_axis_name="c", subcore_axis_name="s")


@functools.partial(
    pl.kernel,
    out_type=jax.ShapeDtypeStruct((EP, U), jnp.float32),
    mesh=_MESH,
    scratch_types=[
        pltpu.VMEM((CPW, CH), jnp.int32),
        pltpu.VMEM((2, GQ * CH, U), jnp.float32),
        pltpu.SemaphoreType.DMA,
        pltpu.SemaphoreType.DMA,
    ],
    compiler_params=_SC_PARAMS,
)
def _sc_gather(x_hbm, idx_hbm, out_hbm, idx_v, bufs, gsem, csem):
    wid = lax.axis_index("s") * NC + lax.axis_index("c")
    pltpu.sync_copy(idx_hbm.at[pl.ds(wid * CPW, CPW)], idx_v)
    couts = [None, None]
    for q in range(NQ):
        buf = bufs.at[q % 2]
        if couts[q % 2] is not None:
            couts[q % 2].wait()
        descs = [
            pltpu.async_copy(
                x_hbm.at[idx_v.at[q * GQ + j]],
                buf.at[pl.ds(j * CH, CH)], gsem)
            for j in range(GQ)
        ]
        for dsc in descs:
            dsc.wait()
        couts[q % 2] = pltpu.async_copy(
            buf, out_hbm.at[pl.ds(wid * PW + q * GQ * CH, GQ * CH)], csem)
    couts[0].wait()
    couts[1].wait()


@functools.partial(
    pl.kernel,
    out_type=jax.ShapeDtypeStruct((NC, NP, U), jnp.float32),
    mesh=_MESH,
    scratch_types=[
        pltpu.VMEM((CPW, CH), jnp.int32),
        pltpu.VMEM((SGRP, CH, U), jnp.float32),
        pltpu.VMEM_SHARED((NP, U), jnp.float32),
        pltpu.SemaphoreType.DMA,
        pltpu.SemaphoreType.DMA,
    ],
    compiler_params=_SC_PARAMS,
)
def _sc_scatter(f_hbm, dst_hbm, zero_hbm, out_hbm, idx_v, bufs, acc,
                lsem, ssem):
    cid = lax.axis_index("c")
    sid = lax.axis_index("s")
    wid = sid * NC + cid
    pltpu.sync_copy(zero_hbm, acc.at[pl.ds(sid * NPT, NPT)])
    plsc.subcore_barrier()
    pltpu.sync_copy(dst_hbm.at[pl.ds(wid * CPW, CPW)], idx_v)

    for g in range(NGRP):
        base = wid * PW + g * SGRP * CH
        loads = [
            pltpu.async_copy(f_hbm.at[pl.ds(base + j * CH, CH)],
                             bufs.at[j], lsem)
            for j in range(SGRP)
        ]
        for dsc in loads:
            dsc.wait()
        scats = [
            pltpu.async_copy(bufs.at[j], acc.at[idx_v.at[g * SGRP + j]],
                             ssem, add=True)
            for j in range(SGRP)
        ]
        for dsc in scats:
            dsc.wait()
    plsc.subcore_barrier()
    pltpu.sync_copy(acc.at[pl.ds(sid * NPT, NPT)],
                    out_hbm.at[cid, pl.ds(sid * NPT, NPT)])


def kernel(node_features, edge_indices, distances, W1, b1, W2, b2, Wt):
    f32 = jnp.float32
    centers = jnp.linspace(0.0, MAXD, G).astype(f32)
    cen_col = jnp.zeros((GP, 1), f32).at[:G, 0].set(centers)
    w1_pad = jnp.zeros((GP, U), f32).at[:G, :].set(W1)
    b1_row = b1.reshape(1, U)
    w2b = W2.reshape(U, U, U).transpose(0, 2, 1).reshape(U * U, U)
    b2m = b2.reshape(U, U).T
    rmat = jnp.repeat(w1_pad, U, axis=1).astype(jnp.bfloat16)
    b1_rep = jnp.repeat(b1_row, U, axis=1)

    src = jnp.concatenate([edge_indices[0],
                           jnp.zeros((EP - E,), jnp.int32)]).reshape(-1, CH)
    dst = jnp.concatenate([edge_indices[1],
                           jnp.zeros((EP - E,), jnp.int32)]).reshape(-1, CH)
    dpad = jnp.concatenate([distances, jnp.zeros((EP - E,), f32)])
    zinit = jnp.zeros((NPT, U), f32)

    x = pl.pallas_call(
        _xform_body,
        out_shape=jax.ShapeDtypeStruct((N, U), f32),
    )(node_features, Wt)

    t_edges = _sc_gather(x, src)

    filtered = pl.pallas_call(
        _edge_body,
        grid=(NBLK,),
        in_specs=[
            pl.BlockSpec((4, EB4), lambda i: (0, i)),
            pl.BlockSpec((EB4, D), lambda i: (i, 0)),
            pl.BlockSpec((GP, 1), lambda i: (0, 0)),
            pl.BlockSpec((GP, U * U), lambda i: (0, 0)),
            pl.BlockSpec((1, U * U), lambda i: (0, 0)),
            pl.BlockSpec((U * U, U), lambda i: (0, 0)),
            pl.BlockSpec((U, U), lambda i: (0, 0)),
        ],
        out_specs=pl.BlockSpec((EB4, D), lambda i: (i, 0)),
        out_shape=jax.ShapeDtypeStruct((EP // 4, D), f32),
    )(dpad.reshape(EP // 4, 4).T, t_edges.reshape(EP // 4, D),
      cen_col, rmat, b1_rep, w2b.astype(jnp.bfloat16), b2m)

    partials = _sc_scatter(filtered.reshape(EP, U), dst, zinit)

    out = pl.pallas_call(
        _final_body,
        out_shape=jax.ShapeDtypeStruct((N, U), f32),
    )(partials[0, :N], partials[1, :N])
    return out


# speedup vs baseline: 4.0095x; 1.0300x over previous
import functools

import jax
import jax.numpy as jnp
from jax import lax
from jax.experimental import pallas as pl
from jax.experimental.pallas import tpu as pltpu
from jax.experimental.pallas import tpu_sc as plsc

N = 10000
E = 160000
D = 128
U = 32
G = 50
GP = 64
GAMMA = 10.0
MAXD = 30.0
LOG2 = 0.6931471805599453

NC = 2
NS = 16
NW = NC * NS
CH = 128
EP = 163840
CPW = EP // (NW * CH)
PW = EP // NW
NP = 10240
NPT = NP // NS

CPW0 = 52
CPW1 = 28
GQ0 = 13
GQ1 = 7
NQ = 4
SGRP = 8
NGRP = CPW // SGRP

EB = 5120
EB4 = EB // 4
NBLK = EP // EB
NREAL = E // EB

_SC_PARAMS = pltpu.CompilerParams(use_tc_tiling_on_sc=False)


def _xform_body(nf_ref, wt_ref, o_ref):
    o_ref[...] = jnp.dot(nf_ref[...], wt_ref[...],
                         preferred_element_type=jnp.float32)


def _edge_body(d_ref, t_ref, cen_ref, r_ref, b1_ref, w2b_ref,
               b2m_ref, o_ref):
    i = pl.program_id(0)
    d4 = d_ref[...]
    cen = cen_ref[...]
    t4 = t_ref[...]
    row4 = i * EB4 + lax.broadcasted_iota(jnp.int32, (EB4, 1), 0)
    mask = row4 < E // 4
    outs = []
    for q in range(4):
        dt = d4[q:q + 1, :]
        diff = dt - cen
        dft = jnp.exp(-GAMMA * diff * diff)
        t = t4[:, q * U:(q + 1) * U]
        tb = t.astype(jnp.bfloat16)
        hrep = (lax.dot_general(dft.astype(jnp.bfloat16), r_ref[...],
                                (((0,), (0,)), ((), ())),
                                preferred_element_type=jnp.float32)
                + b1_ref[...]).astype(jnp.bfloat16)
        ttile = jnp.tile(tb, (1, U))
        p = hrep * ttile
        out = jnp.dot(p, w2b_ref[...], preferred_element_type=jnp.float32)
        out = out + jnp.dot(t, b2m_ref[...],
                            preferred_element_type=jnp.float32)
        outs.append(jnp.where(mask, out, 0.0))
    o_ref[...] = jnp.concatenate(outs, axis=1)


def _final_body(a_ref, b_ref, o_ref):
    x = a_ref[...] + b_ref[...]
    o_ref[...] = (jnp.maximum(x, 0.0)
                  + jnp.log(1.0 + jnp.exp(-jnp.abs(x))) - LOG2)


_MESH = plsc.VectorSubcoreMesh(core_axis_name="c", subcore_axis_name="s")


@functools.partial(
    pl.kernel,
    out_type=jax.ShapeDtypeStruct((EP, U), jnp.float32),
    mesh=_MESH,
    scratch_types=[
        pltpu.VMEM((CPW0, CH), jnp.int32),
        pltpu.VMEM((2, GQ0 * CH, U), jnp.float32),
        pltpu.SemaphoreType.DMA,
        pltpu.SemaphoreType.DMA,
    ],
    compiler_params=_SC_PARAMS,
)
def _sc_gather(x_hbm, idx_hbm, out_hbm, idx_v, bufs, gsem, csem):
    cid = lax.axis_index("c")
    sid = lax.axis_index("s")

    def worker(rowbase, cpw, gq):
        pltpu.sync_copy(idx_hbm.at[pl.ds(rowbase, cpw)],
                        idx_v.at[pl.ds(0, cpw)])
        couts = [None, None]
        for q in range(NQ):
            buf = bufs.at[q % 2]
            if couts[q % 2] is not None:
                couts[q % 2].wait()
            descs = [
                pltpu.async_copy(
                    x_hbm.at[idx_v.at[q * gq + j]],
                    buf.at[pl.ds(j * CH, CH)], gsem)
                for j in range(gq)
            ]
            for dsc in descs:
                dsc.wait()
            couts[q % 2] = pltpu.async_copy(
                buf.at[pl.ds(0, gq * CH)],
                out_hbm.at[pl.ds((rowbase + q * gq) * CH, gq * CH)], csem)
        couts[0].wait()
        couts[1].wait()

    @pl.when(cid == 0)
    def _():
        worker(sid * CPW0, CPW0, GQ0)

    @pl.when(cid == 1)
    def _():
        worker(NS * CPW0 + sid * CPW1, CPW1, GQ1)


@functools.partial(
    pl.kernel,
    out_type=jax.ShapeDtypeStruct((NC, NP, U), jnp.float32),
    mesh=_MESH,
    scratch_types=[
        pltpu.VMEM((CPW, CH), jnp.int32),
        pltpu.VMEM((SGRP, CH, U), jnp.float32),
        pltpu.VMEM_SHARED((NP, U), jnp.float32),
        pltpu.SemaphoreType.DMA,
        pltpu.SemaphoreType.DMA,
    ],
    compiler_params=_SC_PARAMS,
)
def _sc_scatter(f_hbm, dst_hbm, zero_hbm, out_hbm, idx_v, bufs, acc,
                lsem, ssem):
    cid = lax.axis_index("c")
    sid = lax.axis_index("s")
    wid = sid * NC + cid
    pltpu.sync_copy(zero_hbm, acc.at[pl.ds(sid * NPT, NPT)])
    plsc.subcore_barrier()
    pltpu.sync_copy(dst_hbm.at[pl.ds(wid * CPW, CPW)], idx_v)

    for g in range(NGRP):
        base = wid * PW + g * SGRP * CH
        loads = [
            pltpu.async_copy(f_hbm.at[pl.ds(base + j * CH, CH)],
                             bufs.at[j], lsem)
            for j in range(SGRP)
        ]
        for dsc in loads:
            dsc.wait()
        scats = [
            pltpu.async_copy(bufs.at[j], acc.at[idx_v.at[g * SGRP + j]],
                             ssem, add=True)
            for j in range(SGRP)
        ]
        for dsc in scats:
            dsc.wait()
    plsc.subcore_barrier()
    pltpu.sync_copy(acc.at[pl.ds(sid * NPT, NPT)],
                    out_hbm.at[cid, pl.ds(sid * NPT, NPT)])


def kernel(node_features, edge_indices, distances, W1, b1, W2, b2, Wt):
    f32 = jnp.float32
    centers = jnp.linspace(0.0, MAXD, G).astype(f32)
    cen_col = jnp.zeros((GP, 1), f32).at[:G, 0].set(centers)
    w1_pad = jnp.zeros((GP, U), f32).at[:G, :].set(W1)
    b1_row = b1.reshape(1, U)
    w2b = W2.reshape(U, U, U).transpose(0, 2, 1).reshape(U * U, U)
    b2m = b2.reshape(U, U).T
    rmat = jnp.repeat(w1_pad, U, axis=1).astype(jnp.bfloat16)
    b1_rep = jnp.repeat(b1_row, U, axis=1)

    src = jnp.concatenate([edge_indices[0],
                           jnp.zeros((EP - E,), jnp.int32)]).reshape(-1, CH)
    dst = jnp.concatenate([edge_indices[1],
                           jnp.zeros((EP - E,), jnp.int32)]).reshape(-1, CH)
    dpad = jnp.concatenate([distances, jnp.zeros((EP - E,), f32)])
    zinit = jnp.zeros((NPT, U), f32)

    x = pl.pallas_call(
        _xform_body,
        out_shape=jax.ShapeDtypeStruct((N, U), f32),
    )(node_features, Wt)

    t_edges = _sc_gather(x, src)

    filtered = pl.pallas_call(
        _edge_body,
        grid=(NBLK,),
        in_specs=[
            pl.BlockSpec((4, EB4), lambda i: (0, i)),
            pl.BlockSpec((EB4, D), lambda i: (i, 0)),
            pl.BlockSpec((GP, 1), lambda i: (0, 0)),
            pl.BlockSpec((GP, U * U), lambda i: (0, 0)),
            pl.BlockSpec((1, U * U), lambda i: (0, 0)),
            pl.BlockSpec((U * U, U), lambda i: (0, 0)),
            pl.BlockSpec((U, U), lambda i: (0, 0)),
        ],
        out_specs=pl.BlockSpec((EB4, D), lambda i: (i, 0)),
        out_shape=jax.ShapeDtypeStruct((EP // 4, D), f32),
    )(dpad.reshape(EP // 4, 4).T, t_edges.reshape(EP // 4, D),
      cen_col, rmat, b1_rep, w2b.astype(jnp.bfloat16), b2m)

    partials = _sc_scatter(filtered.reshape(EP, U), dst, zinit)

    out = pl.pallas_call(
        _final_body,
        out_shape=jax.ShapeDtypeStruct((N, U), f32),
    )(partials[0, :N], partials[1, :N])
    return out


# speedup vs baseline: 4.0166x; 1.0018x over previous
import functools

import jax
import jax.numpy as jnp
from jax import lax
from jax.experimental import pallas as pl
from jax.experimental.pallas import tpu as pltpu
from jax.experimental.pallas import tpu_sc as plsc

N = 10000
E = 160000
D = 128
U = 32
G = 50
GP = 64
GAMMA = 10.0
MAXD = 30.0
LOG2 = 0.6931471805599453

NC = 2
NS = 16
NW = NC * NS
CH = 128
EP = 163840
CPW = EP // (NW * CH)
PW = EP // NW
NP = 10240
NPT = NP // NS

CPW0 = 52
CPW1 = 28
GQ0 = 13
GQ1 = 7
NQ = 4
SGRP = 8
NGRP = CPW // SGRP

EB = 5120
EB4 = EB // 4
NBLK = EP // EB

_SC_PARAMS = pltpu.CompilerParams(use_tc_tiling_on_sc=False)


def _xform_body(nf_ref, wt_ref, o_ref):
    o_ref[...] = jnp.dot(nf_ref[...], wt_ref[...],
                         preferred_element_type=jnp.float32)


def _edge_body(d_ref, t_ref, cen_ref, r_ref, b1_ref, w2b_ref,
               b2m_ref, o_ref):
    i = pl.program_id(0)
    d4 = d_ref[...]
    cen = cen_ref[...]
    t4 = t_ref[...]
    row4 = i * EB4 + lax.broadcasted_iota(jnp.int32, (EB4, 1), 0)
    mask = row4 < E // 4
    outs = []
    for q in range(4):
        dt = d4[q:q + 1, :]
        diff = dt - cen
        dft = jnp.exp(-GAMMA * diff * diff)
        t = t4[:, q * U:(q + 1) * U]
        tb = t.astype(jnp.bfloat16)
        hrep = (lax.dot_general(dft.astype(jnp.bfloat16), r_ref[...],
                                (((0,), (0,)), ((), ())),
                                preferred_element_type=jnp.float32)
                + b1_ref[...]).astype(jnp.bfloat16)
        ttile = jnp.tile(tb, (1, U))
        p = hrep * ttile
        out = jnp.dot(p, w2b_ref[...], preferred_element_type=jnp.float32)
        out = out + jnp.dot(t, b2m_ref[...],
                            preferred_element_type=jnp.float32)
        outs.append(jnp.where(mask, out, 0.0))
    o_ref[...] = jnp.concatenate(outs, axis=1)


def _final_body(a_ref, b_ref, o_ref):
    x = a_ref[...] + b_ref[...]
    o_ref[...] = (jnp.maximum(x, 0.0)
                  + jnp.log(1.0 + jnp.exp(-jnp.abs(x))) - LOG2)


_MESH = plsc.VectorSubcoreMesh(core_axis_name="c", subcore_axis_name="s")


@functools.partial(
    pl.kernel,
    out_type=jax.ShapeDtypeStruct((EP, U), jnp.float32),
    mesh=_MESH,
    scratch_types=[
        pltpu.VMEM((CPW0, CH), jnp.int32),
        pltpu.VMEM((2, GQ0 * CH, U), jnp.float32),
        pltpu.SemaphoreType.DMA,
        pltpu.SemaphoreType.DMA,
    ],
    compiler_params=_SC_PARAMS,
)
def _sc_gather(x_hbm, idx_hbm, out_hbm, idx_v, bufs, gsem, csem):
    cid = lax.axis_index("c")
    sid = lax.axis_index("s")

    def worker(rowbase, cpw, gq):
        pltpu.sync_copy(idx_hbm.at[pl.ds(rowbase, cpw)],
                        idx_v.at[pl.ds(0, cpw)])
        couts = [None, None]
        for q in range(NQ):
            buf = bufs.at[q % 2]
            if couts[q % 2] is not None:
                couts[q % 2].wait()
            descs = [
                pltpu.async_copy(
                    x_hbm.at[idx_v.at[q * gq + j]],
                    buf.at[pl.ds(j * CH, CH)], gsem)
                for j in range(gq)
            ]
            for dsc in descs:
                dsc.wait()
            couts[q % 2] = pltpu.async_copy(
                buf.at[pl.ds(0, gq * CH)],
                out_hbm.at[pl.ds((rowbase + q * gq) * CH, gq * CH)], csem)
        couts[0].wait()
        couts[1].wait()

    @pl.when(cid == 0)
    def _():
        worker(sid * CPW0, CPW0, GQ0)

    @pl.when(cid == 1)
    def _():
        worker(NS * CPW0 + sid * CPW1, CPW1, GQ1)


@functools.partial(
    pl.kernel,
    out_type=jax.ShapeDtypeStruct((NC, NP, U), jnp.float32),
    mesh=_MESH,
    scratch_types=[
        pltpu.VMEM((CPW, CH), jnp.int32),
        pltpu.VMEM((SGRP, CH, U), jnp.float32),
        pltpu.VMEM_SHARED((NP, U), jnp.float32),
        pltpu.SemaphoreType.DMA,
        pltpu.SemaphoreType.DMA,
    ],
    compiler_params=_SC_PARAMS,
)
def _sc_scatter(f_hbm, dst_hbm, zero_hbm, out_hbm, idx_v, bufs, acc,
                lsem, ssem):
    cid = lax.axis_index("c")
    sid = lax.axis_index("s")
    wid = sid * NC + cid
    pltpu.sync_copy(zero_hbm, acc.at[pl.ds(sid * NPT, NPT)])
    plsc.subcore_barrier()
    pltpu.sync_copy(dst_hbm.at[pl.ds(wid * CPW, CPW)], idx_v)

    for g in range(NGRP):
        base = wid * PW + g * SGRP * CH
        loads = [
            pltpu.async_copy(f_hbm.at[pl.ds(base + j * CH, CH)],
                             bufs.at[j], lsem)
            for j in range(SGRP)
        ]
        for dsc in loads:
            dsc.wait()
        scats = [
            pltpu.async_copy(bufs.at[j], acc.at[idx_v.at[g * SGRP + j]],
                             ssem, add=True)
            for j in range(SGRP)
        ]
        for dsc in scats:
            dsc.wait()
    plsc.subcore_barrier()
    pltpu.sync_copy(acc.at[pl.ds(sid * NPT, NPT)],
                    out_hbm.at[cid, pl.ds(sid * NPT, NPT)])


def kernel(node_features, edge_indices, distances, W1, b1, W2, b2, Wt):
    f32 = jnp.float32
    centers = jnp.linspace(0.0, MAXD, G).astype(f32)
    cen_col = jnp.zeros((GP, 1), f32).at[:G, 0].set(centers)
    w1_pad = jnp.zeros((GP, U), f32).at[:G, :].set(W1)
    b1_row = b1.reshape(1, U)
    w2b = W2.reshape(U, U, U).transpose(0, 2, 1).reshape(U * U, U)
    b2m = b2.reshape(U, U).T
    rmat = jnp.repeat(w1_pad, U, axis=1).astype(jnp.bfloat16)
    b1_rep = jnp.repeat(b1_row, U, axis=1)

    src = jnp.concatenate([edge_indices[0],
                           jnp.zeros((EP - E,), jnp.int32)]).reshape(-1, CH)
    dst = jnp.concatenate([edge_indices[1],
                           jnp.zeros((EP - E,), jnp.int32)]).reshape(-1, CH)
    dpad = jnp.concatenate([distances, jnp.zeros((EP - E,), f32)])
    zinit = jnp.zeros((NPT, U), f32)

    x = pl.pallas_call(
        _xform_body,
        out_shape=jax.ShapeDtypeStruct((N, U), f32),
    )(node_features, Wt)

    t_edges = _sc_gather(x, src)

    filtered = pl.pallas_call(
        _edge_body,
        grid=(NBLK,),
        in_specs=[
            pl.BlockSpec((4, EB4), lambda i: (0, i)),
            pl.BlockSpec((EB4, D), lambda i: (i, 0)),
            pl.BlockSpec((GP, 1), lambda i: (0, 0)),
            pl.BlockSpec((GP, U * U), lambda i: (0, 0)),
            pl.BlockSpec((1, U * U), lambda i: (0, 0)),
            pl.BlockSpec((U * U, U), lambda i: (0, 0)),
            pl.BlockSpec((U, U), lambda i: (0, 0)),
        ],
        out_specs=pl.BlockSpec((EB4, D), lambda i: (i, 0)),
        out_shape=jax.ShapeDtypeStruct((EP // 4, D), f32),
    )(dpad.reshape(EP // 4, 4).T, t_edges.reshape(EP // 4, D),
      cen_col, rmat, b1_rep, w2b.astype(jnp.bfloat16), b2m)

    partials = _sc_scatter(filtered.reshape(EP, U), dst, zinit)

    out = pl.pallas_call(
        _final_body,
        out_shape=jax.ShapeDtypeStruct((N, U), f32),
    )(partials[0, :N], partials[1, :N])
    return out
